# baseline (speedup 1.0000x reference)
import functools

import jax
import jax.numpy as jnp
from jax import lax
from jax.experimental import pallas as pl
from jax.experimental.pallas import tpu as pltpu
from jax.experimental.pallas import tpu_sc as plsc

BATCH = 16384
SEQ = 200
DIM = 16
NPOS = 4096

_NUM_CORES = 2
_NUM_SUBCORES = 16
_NW = _NUM_CORES * _NUM_SUBCORES
_BSTRIPE = BATCH // _NW
_BTILES = _BSTRIPE // 128
_SBLK = 4
_NROUNDS = SEQ // _SBLK


def _sc_encode(pos_t, tbl_t):
    mesh = plsc.VectorSubcoreMesh(core_axis_name="c", subcore_axis_name="s")

    @functools.partial(
        pl.kernel,
        out_type=jax.ShapeDtypeStruct(
            (SEQ, 2, BATCH // 128, 8, 128), jnp.float32
        ),
        mesh=mesh,
        scratch_types=[
            pltpu.VMEM((DIM, NPOS), jnp.float32),
            pltpu.VMEM((2, _SBLK, _BSTRIPE), jnp.int32),
            pltpu.VMEM((_SBLK, 2, _BTILES, 8, 128), jnp.float32),
            pltpu.SemaphoreType.DMA((_SBLK,)),
            pltpu.SemaphoreType.DMA((2,)),
        ],
        compiler_params=pltpu.CompilerParams(needs_layout_passes=False),
    )
    def k(pos_hbm, tbl_hbm, out_hbm, tbl_v, pos_v, stg_v, sem_o, sem_p):
        wid = lax.axis_index("s") * _NUM_CORES + lax.axis_index("c")
        b0 = wid * _BSTRIPE

        def pos_copy(r, pb):
            return pltpu.make_async_copy(
                pos_hbm.at[pl.ds(r * _SBLK, _SBLK), pl.ds(b0, _BSTRIPE)],
                pos_v.at[pb],
                sem_p.at[pb],
            )

        def wait_out(sb):
            for kh in range(2):
                pltpu.make_async_copy(
                    stg_v.at[sb, kh],
                    out_hbm.at[0, kh, pl.ds(wid * _BTILES, _BTILES)],
                    sem_o.at[sb],
                ).wait()

        for pb in range(2):
            pos_copy(pb, pb).start()
        pltpu.sync_copy(tbl_hbm, tbl_v)

        def outer_body(rr, carry):
            for pb in range(2):
                r = 2 * rr + pb
                pos_copy(r, pb).wait()
                for j in range(_SBLK):
                    sb = j
                    s = r * _SBLK + j
                    if pb == 1:
                        wait_out(sb)
                    else:

                        @pl.when(rr > 0)
                        def _():
                            wait_out(sb)

                    @plsc.parallel_loop(0, _BSTRIPE // 16, unroll=8)
                    def g_body(g):
                        p = pos_v[pb, j, pl.ds(g * 16, 16)]
                        bh = g // 8
                        bl = (g % 8) * 16
                        for kk in range(DIM):
                            v = plsc.load_gather(
                                tbl_v, [jnp.full((16,), kk, jnp.int32), p]
                            )
                            stg_v[sb, kk // 8, bh, kk % 8, pl.ds(bl, 16)] = v
                    for kh in range(2):
                        pltpu.async_copy(
                            stg_v.at[sb, kh],
                            out_hbm.at[s, kh, pl.ds(wid * _BTILES, _BTILES)],
                            sem_o.at[sb],
                        )

                @pl.when(r < _NROUNDS - 2)
                def _():
                    pos_copy(r + 2, pb).start()

            return carry

        lax.fori_loop(0, _NROUNDS // 2, outer_body, 0)
        for sb in range(_SBLK):
            wait_out(sb)

    return k(pos_t, tbl_t)


def kernel(positions, position_encoding):
    pos_t = positions.T
    tbl_t = position_encoding.T
    x = _sc_encode(pos_t, tbl_t)
    return x.transpose(2, 4, 0, 1, 3).reshape(BATCH, SEQ, DIM)
